# Initial kernel scaffold; baseline (speedup 1.0000x reference)
import jax
import jax.numpy as jnp
from jax.experimental import pallas as pl


def kernel(all_node_embedding, sub2_row, sub2_col, sub2_left_nodes, sub2_right_common, sub3_row, sub3_col, sub3_left_nodes, sub3_right_specific):
    raise NotImplementedError("write your pallas kernel here")


# speedup vs baseline: 12.7073x; 12.7073x over previous
import functools

import jax
import jax.numpy as jnp
from jax import lax
from jax.experimental import pallas as pl
from jax.experimental.pallas import tpu as pltpu
from jax.experimental.pallas import tpu_sc as plsc

N_ENT = 10000
N_TYPE = 32
D = 64
N_TOT = N_ENT + N_TYPE

N_SUBCORES = 16
LANES = 128
MASK_ELEMS = N_ENT * N_TYPE
ZCHUNK = 1024
ZITERS = 20
TILE_SLICE = ZCHUNK * ZITERS
MASK_PAD = N_SUBCORES * TILE_SLICE
SACRIFICIAL = MASK_PAD - LANES


def _sc_build_masks(flat2, flat3):
    e2 = flat2.shape[0]
    e3 = flat3.shape[0]
    k2 = -(-e2 // (N_SUBCORES * LANES))
    k3 = -(-e3 // (N_SUBCORES * LANES))
    pad2 = N_SUBCORES * k2 * LANES - e2
    pad3 = N_SUBCORES * k3 * LANES - e3
    i32 = jnp.int32
    flat2p = jnp.concatenate(
        [flat2, jnp.full((pad2,), SACRIFICIAL, i32)]).reshape(
            N_SUBCORES, k2, LANES)
    flat3p = jnp.concatenate(
        [flat3, jnp.full((pad3,), SACRIFICIAL, i32)]).reshape(
            N_SUBCORES, k3, LANES)

    mesh = plsc.VectorSubcoreMesh(core_axis_name="c", subcore_axis_name="s")

    @functools.partial(
        pl.kernel,
        mesh=mesh,
        out_type=[
            jax.ShapeDtypeStruct((MASK_PAD,), jnp.float32),
            jax.ShapeDtypeStruct((MASK_PAD,), jnp.float32),
        ],
        scratch_types=[
            pltpu.VMEM((k2, LANES), i32),
            pltpu.VMEM((k3, LANES), i32),
            pltpu.VMEM((LANES,), jnp.float32),
            pltpu.VMEM((ZCHUNK,), jnp.float32),
        ],
    )
    def sc_kernel(flat2_hbm, flat3_hbm, out2_hbm, out3_hbm,
                  idx2_v, idx3_v, ones_v, zeros_v):
        c = lax.axis_index("c")
        s = lax.axis_index("s")

        for i in range(ZCHUNK // 16):
            zeros_v[pl.ds(i * 16, 16)] = jnp.zeros((16,), jnp.float32)
        for i in range(LANES // 16):
            ones_v[pl.ds(i * 16, 16)] = jnp.ones((16,), jnp.float32)

        zbase = s * TILE_SLICE

        @pl.when(c == 0)
        def _():
            for j in range(ZITERS):
                pltpu.sync_copy(
                    zeros_v, out2_hbm.at[pl.ds(zbase + j * ZCHUNK, ZCHUNK)])

        @pl.when(c == 1)
        def _():
            for j in range(ZITERS):
                pltpu.sync_copy(
                    zeros_v, out3_hbm.at[pl.ds(zbase + j * ZCHUNK, ZCHUNK)])

        plsc.subcore_barrier()

        @pl.when(c == 0)
        def _():
            pltpu.sync_copy(flat2_hbm.at[s], idx2_v)
            for j in range(k2):
                pltpu.sync_copy(ones_v, out2_hbm.at[idx2_v.at[j]])

        @pl.when(c == 1)
        def _():
            pltpu.sync_copy(flat3_hbm.at[s], idx3_v)
            for j in range(k3):
                pltpu.sync_copy(ones_v, out3_hbm.at[idx3_v.at[j]])

    out2, out3 = sc_kernel(flat2p, flat3p)
    mask2 = out2[:MASK_ELEMS].reshape(N_ENT, N_TYPE)
    mask3t = out3[:MASK_ELEMS].reshape(N_ENT, N_TYPE)
    return mask2, mask3t


def _tc_body(emb_ref, m2_ref, m3_ref, out_ref):
    ent = emb_ref[0:N_ENT, :]
    typ = emb_ref[N_ENT:N_TOT, :]
    m2 = m2_ref[...]
    m3 = m3_ref[...]

    f32 = jnp.float32
    hi = lax.Precision.HIGHEST
    agg2 = lax.dot_general(m2, ent, (((0,), (0,)), ((), ())),
                           precision=hi, preferred_element_type=f32)
    ones_col = jnp.ones((N_ENT, 1), f32)
    deg2 = lax.dot_general(m2, ones_col, (((0,), (0,)), ((), ())),
                           precision=hi, preferred_element_type=f32)
    type_new = typ + agg2 + (jnp.float32(N_ENT) - deg2)

    s_mat = lax.dot_general(m3, type_new, (((1,), (0,)), ((), ())),
                            precision=hi, preferred_element_type=f32)
    deg3 = jnp.sum(m3, axis=1, keepdims=True)
    out_ent = ent * (1.0 - (s_mat + (jnp.float32(N_TYPE) - deg3))
                     / (1.0 + deg3))

    out_ref[0:N_ENT, :] = out_ent
    out_ref[N_ENT:N_TOT, :] = type_new


def _tc_combine(emb, mask2, mask3t, interpret=False):
    return pl.pallas_call(
        _tc_body,
        out_shape=jax.ShapeDtypeStruct((N_TOT, D), jnp.float32),
        in_specs=[
            pl.BlockSpec(memory_space=pltpu.VMEM),
            pl.BlockSpec(memory_space=pltpu.VMEM),
            pl.BlockSpec(memory_space=pltpu.VMEM),
        ],
        out_specs=pl.BlockSpec(memory_space=pltpu.VMEM),
        interpret=interpret,
    )(emb, mask2, mask3t)


def kernel(all_node_embedding, sub2_row, sub2_col, sub2_left_nodes,
           sub2_right_common, sub3_row, sub3_col, sub3_left_nodes,
           sub3_right_specific):
    i32 = jnp.int32
    flat2 = sub2_row.astype(i32) * N_TYPE + sub2_col.astype(i32)
    flat3 = sub3_col.astype(i32) * N_TYPE + sub3_row.astype(i32)
    mask2, mask3t = _sc_build_masks(flat2, flat3)
    return _tc_combine(all_node_embedding, mask2, mask3t)


# speedup vs baseline: 49.4138x; 3.8886x over previous
import functools

import jax
import jax.numpy as jnp
from jax import lax
from jax.experimental import pallas as pl
from jax.experimental.pallas import tpu as pltpu
from jax.experimental.pallas import tpu_sc as plsc

N_ENT = 10000
N_TYPE = 32
D = 64
N_TOT = N_ENT + N_TYPE

N_SUBCORES = 16
LANES = 128
MASK_ELEMS = N_ENT * N_TYPE
ZCHUNK = 1024
ZITERS = 20
TILE_SLICE = ZCHUNK * ZITERS
MASK_PAD = N_SUBCORES * TILE_SLICE
SACRIFICIAL = MASK_PAD - LANES


def _sc_build_masks(flat2, flat3):
    e2 = flat2.shape[0]
    e3 = flat3.shape[0]
    ep2 = -(-e2 // 16) * 16
    ep3 = -(-e3 // 16) * 16
    i32 = jnp.int32
    flat2p = jnp.concatenate([flat2, jnp.full((ep2 - e2,), SACRIFICIAL, i32)])
    flat3p = jnp.concatenate([flat3, jnp.full((ep3 - e3,), SACRIFICIAL, i32)])

    mesh = plsc.VectorSubcoreMesh(core_axis_name="c", subcore_axis_name="s")

    @functools.partial(
        pl.kernel,
        mesh=mesh,
        out_type=[
            jax.ShapeDtypeStruct((MASK_PAD,), jnp.float32),
            jax.ShapeDtypeStruct((MASK_PAD,), jnp.float32),
        ],
        scratch_types=[
            pltpu.VMEM((ep2,), i32),
            pltpu.VMEM((ep3,), i32),
            pltpu.VMEM((TILE_SLICE,), jnp.float32),
        ],
        compiler_params=pltpu.CompilerParams(needs_layout_passes=False),
    )
    def sc_kernel(flat2_hbm, flat3_hbm, out2_hbm, out3_hbm,
                  idx2_v, idx3_v, mask_v):
        c = lax.axis_index("c")
        s = lax.axis_index("s")
        base = s * TILE_SLICE
        zeros16 = jnp.zeros((16,), jnp.float32)
        ones16 = jnp.ones((16,), jnp.float32)

        def zero_body(i, _):
            mask_v[pl.ds(i * 16, 16)] = zeros16
            return 0

        lax.fori_loop(0, TILE_SLICE // 16, zero_body, 0)

        def scan_body(idx_v):
            def body(i, _):
                v = idx_v[pl.ds(i * 16, 16)]
                rel = v - base
                m = plsc.bitcast(rel, jnp.uint32) < jnp.uint32(TILE_SLICE)
                plsc.store_scatter(mask_v, [rel], ones16, mask=m)
                return 0
            lax.fori_loop(0, idx_v.shape[0] // 16, body, 0)

        @pl.when(c == 0)
        def _():
            pltpu.sync_copy(flat2_hbm, idx2_v)
            scan_body(idx2_v)
            pltpu.sync_copy(mask_v, out2_hbm.at[pl.ds(base, TILE_SLICE)])

        @pl.when(c == 1)
        def _():
            pltpu.sync_copy(flat3_hbm, idx3_v)
            scan_body(idx3_v)
            pltpu.sync_copy(mask_v, out3_hbm.at[pl.ds(base, TILE_SLICE)])

    out2, out3 = sc_kernel(flat2p, flat3p)
    mask2 = out2[:MASK_ELEMS].reshape(N_ENT, N_TYPE)
    mask3t = out3[:MASK_ELEMS].reshape(N_ENT, N_TYPE)
    return mask2, mask3t


def _tc_body(emb_ref, m2_ref, m3_ref, out_ref):
    ent = emb_ref[0:N_ENT, :]
    typ = emb_ref[N_ENT:N_TOT, :]
    m2 = m2_ref[...]
    m3 = m3_ref[...]

    f32 = jnp.float32
    hi = lax.Precision.HIGHEST
    agg2 = lax.dot_general(m2, ent, (((0,), (0,)), ((), ())),
                           precision=hi, preferred_element_type=f32)
    ones_col = jnp.ones((N_ENT, 1), f32)
    deg2 = lax.dot_general(m2, ones_col, (((0,), (0,)), ((), ())),
                           precision=hi, preferred_element_type=f32)
    type_new = typ + agg2 + (jnp.float32(N_ENT) - deg2)

    s_mat = lax.dot_general(m3, type_new, (((1,), (0,)), ((), ())),
                            precision=hi, preferred_element_type=f32)
    deg3 = jnp.sum(m3, axis=1, keepdims=True)
    out_ent = ent * (1.0 - (s_mat + (jnp.float32(N_TYPE) - deg3))
                     / (1.0 + deg3))

    out_ref[0:N_ENT, :] = out_ent
    out_ref[N_ENT:N_TOT, :] = type_new


def _tc_combine(emb, mask2, mask3t, interpret=False):
    return pl.pallas_call(
        _tc_body,
        out_shape=jax.ShapeDtypeStruct((N_TOT, D), jnp.float32),
        in_specs=[
            pl.BlockSpec(memory_space=pltpu.VMEM),
            pl.BlockSpec(memory_space=pltpu.VMEM),
            pl.BlockSpec(memory_space=pltpu.VMEM),
        ],
        out_specs=pl.BlockSpec(memory_space=pltpu.VMEM),
        interpret=interpret,
    )(emb, mask2, mask3t)


def kernel(all_node_embedding, sub2_row, sub2_col, sub2_left_nodes,
           sub2_right_common, sub3_row, sub3_col, sub3_left_nodes,
           sub3_right_specific):
    i32 = jnp.int32
    flat2 = sub2_row.astype(i32) * N_TYPE + sub2_col.astype(i32)
    flat3 = sub3_col.astype(i32) * N_TYPE + sub3_row.astype(i32)
    mask2, mask3t = _sc_build_masks(flat2, flat3)
    return _tc_combine(all_node_embedding, mask2, mask3t)


# speedup vs baseline: 56.7650x; 1.1488x over previous
import functools

import jax
import jax.numpy as jnp
from jax import lax
from jax.experimental import pallas as pl
from jax.experimental.pallas import tpu as pltpu
from jax.experimental.pallas import tpu_sc as plsc

N_ENT = 10000
N_TYPE = 32
D = 64
N_TOT = N_ENT + N_TYPE

N_SUBCORES = 16
MASK_ROWS = 10240
ROWS_PER_TILE = MASK_ROWS // N_SUBCORES
CHUNK = 4096


def _sc_build_masks(sub2_row, sub2_col, sub3_row, sub3_col):
    e2 = sub2_row.shape[0]
    e3 = sub3_row.shape[0]
    i32 = jnp.int32

    mesh = plsc.VectorSubcoreMesh(core_axis_name="c", subcore_axis_name="s")

    @functools.partial(
        pl.kernel,
        mesh=mesh,
        out_type=[
            jax.ShapeDtypeStruct((MASK_ROWS, N_TYPE), jnp.float32),
            jax.ShapeDtypeStruct((MASK_ROWS, N_TYPE), jnp.float32),
        ],
        scratch_types=[
            pltpu.VMEM((CHUNK,), i32),
            pltpu.VMEM((CHUNK,), i32),
            pltpu.VMEM((CHUNK,), i32),
            pltpu.VMEM((CHUNK,), i32),
            pltpu.VMEM((ROWS_PER_TILE, N_TYPE), jnp.float32),
            pltpu.SemaphoreType.DMA,
            pltpu.SemaphoreType.DMA,
            pltpu.SemaphoreType.DMA,
            pltpu.SemaphoreType.DMA,
        ],
        compiler_params=pltpu.CompilerParams(needs_layout_passes=False),
    )
    def sc_kernel(r2_hbm, c2_hbm, r3_hbm, c3_hbm, out2_hbm, out3_hbm,
                  a0_v, a1_v, b0_v, b1_v, mask_v, sa0, sa1, sb0, sb1):
        c = lax.axis_index("c")
        s = lax.axis_index("s")
        base = s * ROWS_PER_TILE
        zeros16 = jnp.zeros((16,), jnp.float32)
        ones16 = jnp.ones((16,), jnp.float32)
        lanes = lax.iota(i32, 16)
        sems_a = (sa0, sa1)
        sems_b = (sb0, sb1)
        bufs_a = (a0_v, a1_v)
        bufs_b = (b0_v, b1_v)

        def zero_body(i, _):
            mask_v[i, pl.ds(0, 16)] = zeros16
            mask_v[i, pl.ds(16, 16)] = zeros16
            return 0

        lax.fori_loop(0, ROWS_PER_TILE, zero_body, 0, unroll=8)

        def scan(n_edges, a_hbm, b_hbm, out_hbm):
            nch = -(-n_edges // CHUNK)
            sizes = [min(CHUNK, n_edges - g * CHUNK) for g in range(nch)]

            def start(g):
                buf = g % 2
                return (
                    pltpu.async_copy(a_hbm.at[pl.ds(g * CHUNK, sizes[g])],
                                     bufs_a[buf].at[pl.ds(0, sizes[g])],
                                     sems_a[buf]),
                    pltpu.async_copy(b_hbm.at[pl.ds(g * CHUNK, sizes[g])],
                                     bufs_b[buf].at[pl.ds(0, sizes[g])],
                                     sems_b[buf]),
                )

            def scan_chunk(g):
                buf = g % 2
                full = sizes[g] // 16
                tail = sizes[g] - full * 16

                def body(i, _):
                    row = bufs_a[buf][pl.ds(i * 16, 16)] - base
                    col = bufs_b[buf][pl.ds(i * 16, 16)]
                    m = (plsc.bitcast(row, jnp.uint32)
                         < jnp.uint32(ROWS_PER_TILE))
                    plsc.store_scatter(mask_v, [row, col], ones16, mask=m)
                    return 0

                lax.fori_loop(0, full, body, 0, unroll=8)
                if tail:
                    row = bufs_a[buf][pl.ds(full * 16, 16)] - base
                    col = bufs_b[buf][pl.ds(full * 16, 16)]
                    m = (plsc.bitcast(row, jnp.uint32)
                         < jnp.uint32(ROWS_PER_TILE))
                    m = m & (lanes < tail)
                    plsc.store_scatter(mask_v, [row, col], ones16, mask=m)

            inflight = {0: start(0)}
            for g in range(nch):
                if g + 1 < nch:
                    inflight[g + 1] = start(g + 1)
                for cp in inflight.pop(g):
                    cp.wait()
                scan_chunk(g)

            pltpu.sync_copy(mask_v, out_hbm.at[pl.ds(base, ROWS_PER_TILE), :])

        @pl.when(c == 0)
        def _():
            scan(e2, r2_hbm, c2_hbm, out2_hbm)

        @pl.when(c == 1)
        def _():
            scan(e3, c3_hbm, r3_hbm, out3_hbm)

    return sc_kernel(sub2_row, sub2_col, sub3_row, sub3_col)


def _tc_body(emb_ref, m2_ref, m3_ref, out_ref):
    ent = emb_ref[0:N_ENT, :]
    typ = emb_ref[N_ENT:N_TOT, :]
    m2 = m2_ref[0:N_ENT, :]
    m3 = m3_ref[0:N_ENT, :]

    f32 = jnp.float32
    hi = lax.Precision.HIGHEST
    ent_aug = jnp.concatenate([ent, jnp.ones((N_ENT, 1), f32)], axis=1)
    agg_aug = lax.dot_general(m2, ent_aug, (((0,), (0,)), ((), ())),
                              precision=hi, preferred_element_type=f32)
    agg2 = agg_aug[:, 0:D]
    deg2 = agg_aug[:, D:D + 1]
    type_new = typ + agg2 + (jnp.float32(N_ENT) - deg2)

    typ_aug = jnp.concatenate([type_new, jnp.ones((N_TYPE, 1), f32)], axis=1)
    s_aug = lax.dot_general(m3, typ_aug, (((1,), (0,)), ((), ())),
                            precision=hi, preferred_element_type=f32)
    s_mat = s_aug[:, 0:D]
    deg3 = s_aug[:, D:D + 1]
    out_ent = ent * (1.0 - (s_mat + (jnp.float32(N_TYPE) - deg3))
                     / (1.0 + deg3))

    out_ref[0:N_ENT, :] = out_ent
    out_ref[N_ENT:N_TOT, :] = type_new


def _tc_combine(emb, mask2, mask3t, interpret=False):
    return pl.pallas_call(
        _tc_body,
        out_shape=jax.ShapeDtypeStruct((N_TOT, D), jnp.float32),
        in_specs=[
            pl.BlockSpec(memory_space=pltpu.VMEM),
            pl.BlockSpec(memory_space=pltpu.VMEM),
            pl.BlockSpec(memory_space=pltpu.VMEM),
        ],
        out_specs=pl.BlockSpec(memory_space=pltpu.VMEM),
        interpret=interpret,
    )(emb, mask2, mask3t)


def kernel(all_node_embedding, sub2_row, sub2_col, sub2_left_nodes,
           sub2_right_common, sub3_row, sub3_col, sub3_left_nodes,
           sub3_right_specific):
    mask2, mask3t = _sc_build_masks(sub2_row, sub2_col, sub3_row, sub3_col)
    return _tc_combine(all_node_embedding, mask2, mask3t)


# speedup vs baseline: 73.5205x; 1.2952x over previous
import functools

import jax
import jax.numpy as jnp
from jax import lax
from jax.experimental import pallas as pl
from jax.experimental.pallas import tpu as pltpu
from jax.experimental.pallas import tpu_sc as plsc

N_ENT = 10000
N_TYPE = 32
D = 64
N_TOT = N_ENT + N_TYPE

N_SUBCORES = 16
MASK_ROWS = 10240
ROWS_PER_TILE = MASK_ROWS // N_SUBCORES
CHUNK = 4096


def _sc_build_masks(sub2_row, sub2_col, sub3_row, sub3_col):
    e2 = sub2_row.shape[0]
    e3 = sub3_row.shape[0]
    i32 = jnp.int32

    mesh = plsc.VectorSubcoreMesh(core_axis_name="c", subcore_axis_name="s")

    @functools.partial(
        pl.kernel,
        mesh=mesh,
        out_type=[
            jax.ShapeDtypeStruct((MASK_ROWS, N_TYPE), jnp.float32),
            jax.ShapeDtypeStruct((MASK_ROWS, N_TYPE), jnp.float32),
        ],
        scratch_types=[
            pltpu.VMEM((CHUNK,), i32),
            pltpu.VMEM((CHUNK,), i32),
            pltpu.VMEM((CHUNK,), i32),
            pltpu.VMEM((CHUNK,), i32),
            pltpu.VMEM((ROWS_PER_TILE, N_TYPE), jnp.float32),
            pltpu.SemaphoreType.DMA,
            pltpu.SemaphoreType.DMA,
            pltpu.SemaphoreType.DMA,
            pltpu.SemaphoreType.DMA,
        ],
        compiler_params=pltpu.CompilerParams(needs_layout_passes=False),
    )
    def sc_kernel(r2_hbm, c2_hbm, r3_hbm, c3_hbm, out2_hbm, out3_hbm,
                  a0_v, a1_v, b0_v, b1_v, mask_v, sa0, sa1, sb0, sb1):
        c = lax.axis_index("c")
        s = lax.axis_index("s")
        base = s * ROWS_PER_TILE
        zeros16 = jnp.zeros((16,), jnp.float32)
        ones16 = jnp.ones((16,), jnp.float32)
        lanes = lax.iota(i32, 16)
        sems_a = (sa0, sa1)
        sems_b = (sb0, sb1)
        bufs_a = (a0_v, a1_v)
        bufs_b = (b0_v, b1_v)

        @plsc.parallel_loop(0, ROWS_PER_TILE, unroll=8)
        def _(i):
            mask_v[i, pl.ds(0, 16)] = zeros16
            mask_v[i, pl.ds(16, 16)] = zeros16

        def scan(n_edges, a_hbm, b_hbm, out_hbm):
            nch = -(-n_edges // CHUNK)
            sizes = [min(CHUNK, n_edges - g * CHUNK) for g in range(nch)]

            def start(g):
                buf = g % 2
                return (
                    pltpu.async_copy(a_hbm.at[pl.ds(g * CHUNK, sizes[g])],
                                     bufs_a[buf].at[pl.ds(0, sizes[g])],
                                     sems_a[buf]),
                    pltpu.async_copy(b_hbm.at[pl.ds(g * CHUNK, sizes[g])],
                                     bufs_b[buf].at[pl.ds(0, sizes[g])],
                                     sems_b[buf]),
                )

            def scan_chunk(g):
                buf = g % 2
                full = sizes[g] // 16
                tail = sizes[g] - full * 16

                @plsc.parallel_loop(0, full * 16, step=16, unroll=8)
                def _(i):
                    row = bufs_a[buf][pl.ds(i, 16)] - base
                    col = bufs_b[buf][pl.ds(i, 16)]
                    m = (plsc.bitcast(row, jnp.uint32)
                         < jnp.uint32(ROWS_PER_TILE))
                    plsc.store_scatter(mask_v, [row, col], ones16, mask=m)
                if tail:
                    row = bufs_a[buf][pl.ds(full * 16, 16)] - base
                    col = bufs_b[buf][pl.ds(full * 16, 16)]
                    m = (plsc.bitcast(row, jnp.uint32)
                         < jnp.uint32(ROWS_PER_TILE))
                    m = m & (lanes < tail)
                    plsc.store_scatter(mask_v, [row, col], ones16, mask=m)

            inflight = {0: start(0)}
            for g in range(nch):
                if g + 1 < nch:
                    inflight[g + 1] = start(g + 1)
                for cp in inflight.pop(g):
                    cp.wait()
                scan_chunk(g)

            pltpu.sync_copy(mask_v, out_hbm.at[pl.ds(base, ROWS_PER_TILE), :])

        @pl.when(c == 0)
        def _():
            scan(e2, r2_hbm, c2_hbm, out2_hbm)

        @pl.when(c == 1)
        def _():
            scan(e3, c3_hbm, r3_hbm, out3_hbm)

    return sc_kernel(sub2_row, sub2_col, sub3_row, sub3_col)


def _tc_body(emb_ref, m2_ref, m3_ref, out_ref):
    ent = emb_ref[0:N_ENT, :]
    typ = emb_ref[N_ENT:N_TOT, :]
    m2 = m2_ref[0:N_ENT, :]
    m3 = m3_ref[0:N_ENT, :]

    f32 = jnp.float32
    ent_aug = jnp.concatenate([ent, jnp.ones((N_ENT, 1), f32)], axis=1)
    agg_aug = lax.dot_general(m2, ent_aug, (((0,), (0,)), ((), ())),
                              preferred_element_type=f32)
    agg2 = agg_aug[:, 0:D]
    deg2 = agg_aug[:, D:D + 1]
    type_new = typ + agg2 + (jnp.float32(N_ENT) - deg2)

    typ_aug = jnp.concatenate([type_new, jnp.ones((N_TYPE, 1), f32)], axis=1)
    s_aug = lax.dot_general(m3, typ_aug, (((1,), (0,)), ((), ())),
                            preferred_element_type=f32)
    s_mat = s_aug[:, 0:D]
    deg3 = s_aug[:, D:D + 1]
    out_ent = ent * (1.0 - (s_mat + (jnp.float32(N_TYPE) - deg3))
                     / (1.0 + deg3))

    out_ref[0:N_ENT, :] = out_ent
    out_ref[N_ENT:N_TOT, :] = type_new


def _tc_combine(emb, mask2, mask3t, interpret=False):
    return pl.pallas_call(
        _tc_body,
        out_shape=jax.ShapeDtypeStruct((N_TOT, D), jnp.float32),
        in_specs=[
            pl.BlockSpec(memory_space=pltpu.VMEM),
            pl.BlockSpec(memory_space=pltpu.VMEM),
            pl.BlockSpec(memory_space=pltpu.VMEM),
        ],
        out_specs=pl.BlockSpec(memory_space=pltpu.VMEM),
        interpret=interpret,
    )(emb, mask2, mask3t)


def kernel(all_node_embedding, sub2_row, sub2_col, sub2_left_nodes,
           sub2_right_common, sub3_row, sub3_col, sub3_left_nodes,
           sub3_right_specific):
    mask2, mask3t = _sc_build_masks(sub2_row, sub2_col, sub3_row, sub3_col)
    return _tc_combine(all_node_embedding, mask2, mask3t)


# speedup vs baseline: 89.4289x; 1.2164x over previous
import functools

import jax
import jax.numpy as jnp
from jax import lax
from jax.experimental import pallas as pl
from jax.experimental.pallas import tpu as pltpu
from jax.experimental.pallas import tpu_sc as plsc

N_ENT = 10000
N_TYPE = 32
D = 64
N_TOT = N_ENT + N_TYPE

N_SUBCORES = 16
MASK_ROWS = 10240
ROWS_PER_TILE = MASK_ROWS // N_SUBCORES
CHUNK = 4096


def _sc_build_masks(sub2_row, sub2_col, sub3_row, sub3_col):
    e2 = sub2_row.shape[0]
    e3 = sub3_row.shape[0]
    i32 = jnp.int32

    mesh = plsc.VectorSubcoreMesh(core_axis_name="c", subcore_axis_name="s")

    @functools.partial(
        pl.kernel,
        mesh=mesh,
        out_type=[
            jax.ShapeDtypeStruct((MASK_ROWS, N_TYPE), jnp.float32),
            jax.ShapeDtypeStruct((MASK_ROWS, N_TYPE), jnp.float32),
        ],
        scratch_types=[
            pltpu.VMEM((CHUNK,), i32),
            pltpu.VMEM((CHUNK,), i32),
            pltpu.VMEM((CHUNK,), i32),
            pltpu.VMEM((CHUNK,), i32),
            pltpu.VMEM((ROWS_PER_TILE, N_TYPE), jnp.float32),
            pltpu.SemaphoreType.DMA,
            pltpu.SemaphoreType.DMA,
            pltpu.SemaphoreType.DMA,
            pltpu.SemaphoreType.DMA,
        ],
        compiler_params=pltpu.CompilerParams(needs_layout_passes=False),
    )
    def sc_kernel(r2_hbm, c2_hbm, r3_hbm, c3_hbm, out2_hbm, out3_hbm,
                  a0_v, a1_v, b0_v, b1_v, mask_v, sa0, sa1, sb0, sb1):
        c = lax.axis_index("c")
        s = lax.axis_index("s")
        base = s * ROWS_PER_TILE
        zeros16 = jnp.zeros((16,), jnp.float32)
        ones16 = jnp.ones((16,), jnp.float32)
        lanes = lax.iota(i32, 16)
        sems_a = (sa0, sa1)
        sems_b = (sb0, sb1)
        bufs_a = (a0_v, a1_v)
        bufs_b = (b0_v, b1_v)

        @plsc.parallel_loop(0, ROWS_PER_TILE, unroll=8)
        def _(i):
            mask_v[i, pl.ds(0, 16)] = zeros16
            mask_v[i, pl.ds(16, 16)] = zeros16

        def scan(n_edges, a_hbm, b_hbm, out_hbm):
            nch = -(-n_edges // CHUNK)
            sizes = [min(CHUNK, n_edges - g * CHUNK) for g in range(nch)]

            def start(g):
                buf = g % 2
                return (
                    pltpu.async_copy(a_hbm.at[pl.ds(g * CHUNK, sizes[g])],
                                     bufs_a[buf].at[pl.ds(0, sizes[g])],
                                     sems_a[buf]),
                    pltpu.async_copy(b_hbm.at[pl.ds(g * CHUNK, sizes[g])],
                                     bufs_b[buf].at[pl.ds(0, sizes[g])],
                                     sems_b[buf]),
                )

            def scan_chunk(g):
                buf = g % 2
                full = sizes[g] // 16
                tail = sizes[g] - full * 16

                @plsc.parallel_loop(0, full * 16, step=16, unroll=8)
                def _(i):
                    row = bufs_a[buf][pl.ds(i, 16)] - base
                    col = bufs_b[buf][pl.ds(i, 16)]
                    m = (plsc.bitcast(row, jnp.uint32)
                         < jnp.uint32(ROWS_PER_TILE))
                    plsc.store_scatter(mask_v, [row, col], ones16, mask=m)
                if tail:
                    row = bufs_a[buf][pl.ds(full * 16, 16)] - base
                    col = bufs_b[buf][pl.ds(full * 16, 16)]
                    m = (plsc.bitcast(row, jnp.uint32)
                         < jnp.uint32(ROWS_PER_TILE))
                    m = m & (lanes < tail)
                    plsc.store_scatter(mask_v, [row, col], ones16, mask=m)

            inflight = {0: start(0)}
            for g in range(nch):
                if g + 1 < nch:
                    inflight[g + 1] = start(g + 1)
                for cp in inflight.pop(g):
                    cp.wait()
                scan_chunk(g)

            pltpu.sync_copy(mask_v, out_hbm.at[pl.ds(base, ROWS_PER_TILE), :])

        @pl.when(c == 0)
        def _():
            scan(e2, r2_hbm, c2_hbm, out2_hbm)

        @pl.when(c == 1)
        def _():
            scan(e3, c3_hbm, r3_hbm, out3_hbm)

    return sc_kernel(sub2_row, sub2_col, sub3_row, sub3_col)


def _tc_body(embt_ref, m2_ref, m3_ref, outt_ref):
    ent = embt_ref[:, 0:N_ENT]
    typ = embt_ref[:, N_ENT:N_TOT]
    m2 = m2_ref[0:N_ENT, :]
    m3 = m3_ref[0:N_ENT, :]

    f32 = jnp.float32
    ent_aug = jnp.concatenate([ent, jnp.ones((1, N_ENT), f32)], axis=0)
    agg_aug = lax.dot_general(ent_aug, m2, (((1,), (0,)), ((), ())),
                              preferred_element_type=f32)
    agg2 = agg_aug[0:D, :]
    deg2 = agg_aug[D:D + 1, :]
    type_new = typ + agg2 + (jnp.float32(N_ENT) - deg2)

    typ_aug = jnp.concatenate([type_new, jnp.ones((1, N_TYPE), f32)], axis=0)
    s_aug = lax.dot_general(typ_aug, m3, (((1,), (1,)), ((), ())),
                            preferred_element_type=f32)
    s_mat = s_aug[0:D, :]
    deg3 = s_aug[D:D + 1, :]
    out_ent = ent * (1.0 - (s_mat + (jnp.float32(N_TYPE) - deg3))
                     / (1.0 + deg3))

    outt_ref[:, 0:N_ENT] = out_ent
    outt_ref[:, N_ENT:N_TOT] = type_new


def _tc_combine(embt, mask2, mask3t, interpret=False):
    return pl.pallas_call(
        _tc_body,
        out_shape=jax.ShapeDtypeStruct((D, N_TOT), jnp.float32),
        in_specs=[
            pl.BlockSpec(memory_space=pltpu.VMEM),
            pl.BlockSpec(memory_space=pltpu.VMEM),
            pl.BlockSpec(memory_space=pltpu.VMEM),
        ],
        out_specs=pl.BlockSpec(memory_space=pltpu.VMEM),
        interpret=interpret,
    )(embt, mask2, mask3t)


def kernel(all_node_embedding, sub2_row, sub2_col, sub2_left_nodes,
           sub2_right_common, sub3_row, sub3_col, sub3_left_nodes,
           sub3_right_specific):
    mask2, mask3t = _sc_build_masks(sub2_row, sub2_col, sub3_row, sub3_col)
    outt = _tc_combine(all_node_embedding.T, mask2, mask3t)
    return outt.T
